# baseline (speedup 1.0000x reference)
import jax
import jax.numpy as jnp
from jax import lax
from jax.experimental import pallas as pl
from jax.experimental.pallas import tpu as pltpu
from jax.experimental.pallas import tpu_sc as plsc

N_ROWS = 16384
N_COLS = 431
MISC_W = 15
CARD_W = 416
SEG_A = 156

NUM_CORES = 2
NUM_SUBCORES = 16
NUM_WORKERS = NUM_CORES * NUM_SUBCORES
ROWS_PER_WORKER = N_ROWS // NUM_WORKERS
CHUNK = 64
N_CHUNKS = ROWS_PER_WORKER // CHUNK
LANES = 16


def _split_kernel(in_hbm, misc_hbm, cards_hbm, in_v, misc_v, cards_v):
    wid = lax.axis_index("s") * NUM_CORES + lax.axis_index("c")

    lane = lax.iota(jnp.int32, LANES)
    msrc = jnp.where(lane < 5, lane, lane + 156)
    mmask = lane < MISC_W

    def src_col(c):
        return c + 5 if c < SEG_A else c + 15

    def row_body(r):
        rvec = jnp.full((LANES,), r, jnp.int32)
        for k in range(CARD_W // LANES):
            out0 = LANES * k
            srcs = [src_col(out0 + i) for i in range(LANES)]
            contig = srcs == list(range(srcs[0], srcs[0] + LANES))
            in_tile = srcs[0] % 128 <= 112
            if contig and in_tile:
                cards_v[r, pl.ds(out0, 16)] = in_v[r, pl.ds(srcs[0], 16)]
            else:
                col = out0 + lane
                svec = col + jnp.where(col < SEG_A, 5, 15)
                cards_v[r, pl.ds(out0, 16)] = plsc.load_gather(in_v, [rvec, svec])
        m = plsc.load_gather(in_v, [rvec, msrc], mask=mmask)
        plsc.store_scatter(misc_v, [rvec, lane], m, mask=mmask)

    for j in range(N_CHUNKS):
        base = wid * ROWS_PER_WORKER + j * CHUNK
        rows = pl.ds(base, CHUNK)
        pltpu.sync_copy(in_hbm.at[rows, :], in_v)
        plsc.parallel_loop(0, CHUNK, unroll=2)(row_body)
        pltpu.sync_copy(misc_v, misc_hbm.at[rows, :])
        pltpu.sync_copy(cards_v, cards_hbm.at[rows, :])


@jax.jit
def kernel(inputs):
    mesh = plsc.VectorSubcoreMesh(core_axis_name="c", subcore_axis_name="s")
    run = pl.kernel(
        _split_kernel,
        out_type=(
            jax.ShapeDtypeStruct((N_ROWS, MISC_W), jnp.float32),
            jax.ShapeDtypeStruct((N_ROWS, CARD_W), jnp.float32),
        ),
        mesh=mesh,
        scratch_types=[
            pltpu.VMEM((CHUNK, N_COLS), jnp.float32),
            pltpu.VMEM((CHUNK, MISC_W), jnp.float32),
            pltpu.VMEM((CHUNK, CARD_W), jnp.float32),
        ],
        compiler_params=pltpu.CompilerParams(needs_layout_passes=False),
    )
    return run(inputs)


# speedup vs baseline: 2.3937x; 2.3937x over previous
import jax
import jax.numpy as jnp
from jax import lax
from jax.experimental import pallas as pl
from jax.experimental.pallas import tpu as pltpu
from jax.experimental.pallas import tpu_sc as plsc

N_ROWS = 16384
N_COLS = 431
MISC_W = 15
CARD_W = 416
SEG_A = 156

NUM_CORES = 2
NUM_SUBCORES = 16
NUM_WORKERS = NUM_CORES * NUM_SUBCORES
LANES = 16

COLS_PER_TILE = N_ROWS // NUM_WORKERS
CHUNK = 128
N_CHUNKS = COLS_PER_TILE // CHUNK
VECS = CHUNK // LANES

MISC_SRC = tuple(range(5)) + tuple(range(161, 171))


def _split_kernel(x_hbm, misc_hbm, cards_hbm, xbuf, cbuf, mbuf, sem):
    tid = lax.axis_index("s") * NUM_CORES + lax.axis_index("c")

    for i in range(N_CHUNKS):
        col0 = pl.multiple_of(tid * COLS_PER_TILE + i * CHUNK, CHUNK)
        pltpu.async_copy(x_hbm.at[:, pl.ds(col0, CHUNK)], xbuf, sem).wait()

        @plsc.parallel_loop(0, SEG_A, 1, unroll=4)
        def _seg_a(r):
            for k in range(VECS):
                cbuf[r, pl.ds(k * LANES, LANES)] = xbuf[r + 5, pl.ds(k * LANES, LANES)]

        @plsc.parallel_loop(SEG_A, CARD_W, 1, unroll=4)
        def _seg_b(r):
            for k in range(VECS):
                cbuf[r, pl.ds(k * LANES, LANES)] = xbuf[r + 15, pl.ds(k * LANES, LANES)]

        for r, src in enumerate(MISC_SRC):
            for k in range(VECS):
                mbuf[r, pl.ds(k * LANES, LANES)] = xbuf[src, pl.ds(k * LANES, LANES)]

        pltpu.async_copy(cbuf, cards_hbm.at[:, pl.ds(col0, CHUNK)], sem).wait()
        pltpu.async_copy(mbuf, misc_hbm.at[:, pl.ds(col0, CHUNK)], sem).wait()


@jax.jit
def kernel(inputs):
    x = inputs.T
    mesh = plsc.VectorSubcoreMesh(core_axis_name="c", subcore_axis_name="s")
    run = pl.kernel(
        _split_kernel,
        out_type=(
            jax.ShapeDtypeStruct((MISC_W, N_ROWS), jnp.float32),
            jax.ShapeDtypeStruct((CARD_W, N_ROWS), jnp.float32),
        ),
        mesh=mesh,
        scratch_types=[
            pltpu.VMEM((N_COLS, CHUNK), jnp.float32),
            pltpu.VMEM((CARD_W, CHUNK), jnp.float32),
            pltpu.VMEM((MISC_W, CHUNK), jnp.float32),
            pltpu.SemaphoreType.DMA,
        ],
        compiler_params=pltpu.CompilerParams(needs_layout_passes=False),
    )
    misc_t, cards_t = run(x)
    return misc_t.T, cards_t.T


# speedup vs baseline: 2.4094x; 1.0065x over previous
import jax
import jax.numpy as jnp
from jax import lax
from jax.experimental import pallas as pl
from jax.experimental.pallas import tpu as pltpu
from jax.experimental.pallas import tpu_sc as plsc

N_ROWS = 16384
N_COLS = 431
MISC_W = 15
CARD_W = 416
SEG_A = 156

NUM_CORES = 2
NUM_SUBCORES = 16
NUM_WORKERS = NUM_CORES * NUM_SUBCORES
LANES = 16

COLS_PER_TILE = N_ROWS // NUM_WORKERS
CHUNK = 128
N_CHUNKS = COLS_PER_TILE // CHUNK
VECS = CHUNK // LANES

MISC_SRC = tuple(range(5)) + tuple(range(161, 171))


def _split_kernel(x_hbm, misc_hbm, cards_hbm, xbuf, cbuf, mbuf, sem_in, sem_c, sem_m):
    tid = lax.axis_index("s") * NUM_CORES + lax.axis_index("c")

    def col_at(i):
        return pl.multiple_of(tid * COLS_PER_TILE + i * CHUNK, CHUNK)

    h_in = pltpu.async_copy(x_hbm.at[:, pl.ds(col_at(0), CHUNK)], xbuf, sem_in)
    h_c = h_m = None

    for i in range(N_CHUNKS):
        col0 = col_at(i)
        h_in.wait()
        if h_c is not None:
            h_c.wait()
            h_m.wait()

        @plsc.parallel_loop(0, SEG_A, 1, unroll=4)
        def _seg_a(r):
            for k in range(VECS):
                cbuf[r, pl.ds(k * LANES, LANES)] = xbuf[r + 5, pl.ds(k * LANES, LANES)]

        @plsc.parallel_loop(SEG_A, CARD_W, 1, unroll=4)
        def _seg_b(r):
            for k in range(VECS):
                cbuf[r, pl.ds(k * LANES, LANES)] = xbuf[r + 15, pl.ds(k * LANES, LANES)]

        for r, src in enumerate(MISC_SRC):
            for k in range(VECS):
                mbuf[r, pl.ds(k * LANES, LANES)] = xbuf[src, pl.ds(k * LANES, LANES)]

        h_c = pltpu.async_copy(cbuf, cards_hbm.at[:, pl.ds(col0, CHUNK)], sem_c)
        h_m = pltpu.async_copy(mbuf, misc_hbm.at[:, pl.ds(col0, CHUNK)], sem_m)
        if i + 1 < N_CHUNKS:
            h_in = pltpu.async_copy(x_hbm.at[:, pl.ds(col_at(i + 1), CHUNK)], xbuf, sem_in)

    h_c.wait()
    h_m.wait()


@jax.jit
def kernel(inputs):
    x = inputs.T
    mesh = plsc.VectorSubcoreMesh(core_axis_name="c", subcore_axis_name="s")
    run = pl.kernel(
        _split_kernel,
        out_type=(
            jax.ShapeDtypeStruct((MISC_W, N_ROWS), jnp.float32),
            jax.ShapeDtypeStruct((CARD_W, N_ROWS), jnp.float32),
        ),
        mesh=mesh,
        scratch_types=[
            pltpu.VMEM((N_COLS, CHUNK), jnp.float32),
            pltpu.VMEM((CARD_W, CHUNK), jnp.float32),
            pltpu.VMEM((MISC_W, CHUNK), jnp.float32),
            pltpu.SemaphoreType.DMA,
            pltpu.SemaphoreType.DMA,
            pltpu.SemaphoreType.DMA,
        ],
        compiler_params=pltpu.CompilerParams(needs_layout_passes=False),
    )
    misc_t, cards_t = run(x)
    return misc_t.T, cards_t.T


# speedup vs baseline: 2.7446x; 1.1391x over previous
import jax
import jax.numpy as jnp
from jax import lax
from jax.experimental import pallas as pl
from jax.experimental.pallas import tpu as pltpu
from jax.experimental.pallas import tpu_sc as plsc

N_ROWS = 16384
N_COLS = 431
MISC_W = 15
CARD_W = 416
SEG_A = 156

NUM_CORES = 2
NUM_SUBCORES = 16
NUM_WORKERS = NUM_CORES * NUM_SUBCORES
LANES = 16

COLS_PER_TILE = N_ROWS // NUM_WORKERS
CHUNK = 128
N_CHUNKS = COLS_PER_TILE // CHUNK
VECS = CHUNK // LANES

CARD_SPLIT = 208
A_ROWS = 224
B_OFF = 216
B_ROWS = N_COLS - B_OFF
B_SHIFT = 15 - (B_OFF - CARD_SPLIT)

MISC_SRC = tuple(range(5)) + tuple(range(161, 171))


def _split_kernel(x_hbm, misc_hbm, cards_hbm,
                  xbufA, xbufB, cbufA, cbufB, mbuf,
                  s_inA, s_inB, s_cA, s_cB, s_m):
    tid = lax.axis_index("s") * NUM_CORES + lax.axis_index("c")

    def col_at(i):
        return pl.multiple_of(tid * COLS_PER_TILE + i * CHUNK, CHUNK)

    def in_a(i):
        return pltpu.async_copy(
            x_hbm.at[pl.ds(0, A_ROWS), pl.ds(col_at(i), CHUNK)], xbufA, s_inA)

    def in_b(i):
        return pltpu.async_copy(
            x_hbm.at[pl.ds(B_OFF, B_ROWS), pl.ds(col_at(i), CHUNK)], xbufB, s_inB)

    h_inA = in_a(0)
    h_inB = in_b(0)
    h_cA = h_cB = h_m = None

    for i in range(N_CHUNKS):
        col0 = col_at(i)

        h_inA.wait()
        if h_cA is not None:
            h_cA.wait()
            h_m.wait()

        @plsc.parallel_loop(0, SEG_A, 1, unroll=4)
        def _seg_a1(r):
            for k in range(VECS):
                cbufA[r, pl.ds(k * LANES, LANES)] = xbufA[r + 5, pl.ds(k * LANES, LANES)]

        @plsc.parallel_loop(SEG_A, CARD_SPLIT, 1, unroll=4)
        def _seg_a2(r):
            for k in range(VECS):
                cbufA[r, pl.ds(k * LANES, LANES)] = xbufA[r + 15, pl.ds(k * LANES, LANES)]

        for r, src in enumerate(MISC_SRC):
            for k in range(VECS):
                mbuf[r, pl.ds(k * LANES, LANES)] = xbufA[src, pl.ds(k * LANES, LANES)]

        h_cA = pltpu.async_copy(
            cbufA, cards_hbm.at[pl.ds(0, CARD_SPLIT), pl.ds(col0, CHUNK)], s_cA)
        h_m = pltpu.async_copy(mbuf, misc_hbm.at[:, pl.ds(col0, CHUNK)], s_m)
        if i + 1 < N_CHUNKS:
            h_inA = in_a(i + 1)

        h_inB.wait()
        if h_cB is not None:
            h_cB.wait()

        @plsc.parallel_loop(0, CARD_W - CARD_SPLIT, 1, unroll=4)
        def _seg_b(r):
            for k in range(VECS):
                cbufB[r, pl.ds(k * LANES, LANES)] = xbufB[r + B_SHIFT, pl.ds(k * LANES, LANES)]

        h_cB = pltpu.async_copy(
            cbufB, cards_hbm.at[pl.ds(CARD_SPLIT, CARD_W - CARD_SPLIT), pl.ds(col0, CHUNK)],
            s_cB)
        if i + 1 < N_CHUNKS:
            h_inB = in_b(i + 1)

    h_cA.wait()
    h_m.wait()
    h_cB.wait()


@jax.jit
def kernel(inputs):
    x = inputs.T
    mesh = plsc.VectorSubcoreMesh(core_axis_name="c", subcore_axis_name="s")
    run = pl.kernel(
        _split_kernel,
        out_type=(
            jax.ShapeDtypeStruct((MISC_W, N_ROWS), jnp.float32),
            jax.ShapeDtypeStruct((CARD_W, N_ROWS), jnp.float32),
        ),
        mesh=mesh,
        scratch_types=[
            pltpu.VMEM((A_ROWS, CHUNK), jnp.float32),
            pltpu.VMEM((B_ROWS, CHUNK), jnp.float32),
            pltpu.VMEM((CARD_SPLIT, CHUNK), jnp.float32),
            pltpu.VMEM((CARD_W - CARD_SPLIT, CHUNK), jnp.float32),
            pltpu.VMEM((MISC_W, CHUNK), jnp.float32),
            pltpu.SemaphoreType.DMA,
            pltpu.SemaphoreType.DMA,
            pltpu.SemaphoreType.DMA,
            pltpu.SemaphoreType.DMA,
            pltpu.SemaphoreType.DMA,
        ],
        compiler_params=pltpu.CompilerParams(needs_layout_passes=False),
    )
    misc_t, cards_t = run(x)
    return misc_t.T, cards_t.T


# speedup vs baseline: 2.9132x; 1.0615x over previous
import jax
import jax.numpy as jnp
from jax import lax
from jax.experimental import pallas as pl
from jax.experimental.pallas import tpu as pltpu
from jax.experimental.pallas import tpu_sc as plsc

N_ROWS = 16384
N_COLS = 431
MISC_W = 15
CARD_W = 416

NUM_CORES = 2
NUM_SUBCORES = 16
NUM_WORKERS = NUM_CORES * NUM_SUBCORES
LANES = 16

COLS_PER_TILE = N_ROWS // NUM_WORKERS
CHUNK = 128
N_CHUNKS = COLS_PER_TILE // CHUNK
VECS = CHUNK // LANES

BRIDGE_SRC = (157, 158, 159, 160, 171, 172, 173, 174)
MISC_SRC = tuple(range(5)) + tuple(range(161, 171))


def _split_kernel(x_hbm, misc_hbm, cards_hbm,
                  xb0, xb1, bb0, bb1, mb0, mb1,
                  s_in0, s_in1, s_o0, s_o1):
    tid = lax.axis_index("s") * NUM_CORES + lax.axis_index("c")

    def col_at(i):
        return pl.multiple_of(tid * COLS_PER_TILE + i * CHUNK, CHUNK)

    xbufs = (xb0, xb1)
    bbufs = (bb0, bb1)
    mbufs = (mb0, mb1)
    s_in = (s_in0, s_in1)
    s_out = (s_o0, s_o1)

    def stage_in(i, p):
        return pltpu.async_copy(
            x_hbm.at[:, pl.ds(col_at(i), CHUNK)], xbufs[p], s_in[p])

    h_in = [stage_in(0, 0), stage_in(1, 1)]
    h_out = [None, None]

    for i in range(N_CHUNKS):
        p = i % 2
        q = 1 - p
        col0 = col_at(i)
        xbuf, bbuf, mbuf = xbufs[p], bbufs[p], mbufs[p]
        h_in[p].wait()

        for r, src in enumerate(BRIDGE_SRC):
            for k in range(VECS):
                bbuf[r, pl.ds(k * LANES, LANES)] = xbuf[src, pl.ds(k * LANES, LANES)]
        for r, src in enumerate(MISC_SRC):
            for k in range(VECS):
                mbuf[r, pl.ds(k * LANES, LANES)] = xbuf[src, pl.ds(k * LANES, LANES)]

        dst_cols = pl.ds(col0, CHUNK)
        h_out[p] = (
            pltpu.async_copy(
                xbuf.at[pl.ds(5, 152)], cards_hbm.at[pl.ds(0, 152), dst_cols], s_out[p]),
            pltpu.async_copy(
                xbuf.at[pl.ds(175, 256)], cards_hbm.at[pl.ds(160, 256), dst_cols], s_out[p]),
            pltpu.async_copy(bbuf, cards_hbm.at[pl.ds(152, 8), dst_cols], s_out[p]),
            pltpu.async_copy(mbuf, misc_hbm.at[:, dst_cols], s_out[p]),
        )
        if 2 <= i + 1 < N_CHUNKS:
            for h in h_out[q]:
                h.wait()
            h_in[q] = stage_in(i + 1, q)

    for p in range(2):
        for h in h_out[p]:
            h.wait()


@jax.jit
def kernel(inputs):
    x = inputs.T
    mesh = plsc.VectorSubcoreMesh(core_axis_name="c", subcore_axis_name="s")
    run = pl.kernel(
        _split_kernel,
        out_type=(
            jax.ShapeDtypeStruct((MISC_W, N_ROWS), jnp.float32),
            jax.ShapeDtypeStruct((CARD_W, N_ROWS), jnp.float32),
        ),
        mesh=mesh,
        scratch_types=[
            pltpu.VMEM((N_COLS, CHUNK), jnp.float32),
            pltpu.VMEM((N_COLS, CHUNK), jnp.float32),
            pltpu.VMEM((8, CHUNK), jnp.float32),
            pltpu.VMEM((8, CHUNK), jnp.float32),
            pltpu.VMEM((MISC_W, CHUNK), jnp.float32),
            pltpu.VMEM((MISC_W, CHUNK), jnp.float32),
            pltpu.SemaphoreType.DMA,
            pltpu.SemaphoreType.DMA,
            pltpu.SemaphoreType.DMA,
            pltpu.SemaphoreType.DMA,
        ],
        compiler_params=pltpu.CompilerParams(needs_layout_passes=False),
    )
    misc_t, cards_t = run(x)
    return misc_t.T, cards_t.T


# speedup vs baseline: 2.9381x; 1.0085x over previous
import jax
import jax.numpy as jnp
from jax import lax
from jax.experimental import pallas as pl
from jax.experimental.pallas import tpu as pltpu
from jax.experimental.pallas import tpu_sc as plsc

N_ROWS = 16384
N_COLS = 431
MISC_W = 15
CARD_W = 416

NUM_CORES = 2
NUM_SUBCORES = 16
NUM_WORKERS = NUM_CORES * NUM_SUBCORES
LANES = 16

COLS_PER_TILE = N_ROWS // NUM_WORKERS
CHUNK = 128
N_CHUNKS = COLS_PER_TILE // CHUNK
VECS = CHUNK // LANES

BRIDGE_SRC = (157, 158, 159, 160, 171, 172, 173, 174)
MISC_SRC = tuple(range(5)) + tuple(range(161, 171))


HALF = 216


def _split_kernel(x_hbm, misc_hbm, cards_hbm,
                  xb0, xb1, bb0, bb1, mb0, mb1,
                  s_a0, s_a1, s_b0, s_b1, s_o0, s_o1):
    tid = lax.axis_index("s") * NUM_CORES + lax.axis_index("c")

    def col_at(i):
        return pl.multiple_of(tid * COLS_PER_TILE + i * CHUNK, CHUNK)

    xbufs = (xb0, xb1)
    bbufs = (bb0, bb1)
    mbufs = (mb0, mb1)
    s_a = (s_a0, s_a1)
    s_b = (s_b0, s_b1)
    s_out = (s_o0, s_o1)

    def stage_in(i, p):
        cols = pl.ds(col_at(i), CHUNK)
        return (
            pltpu.async_copy(
                x_hbm.at[pl.ds(0, HALF), cols], xbufs[p].at[pl.ds(0, HALF)], s_a[p]),
            pltpu.async_copy(
                x_hbm.at[pl.ds(HALF, N_COLS - HALF), cols],
                xbufs[p].at[pl.ds(HALF, N_COLS - HALF)], s_b[p]),
        )

    h_in = [stage_in(0, 0), stage_in(1, 1)]
    h_out = [None, None]

    for i in range(N_CHUNKS):
        p = i % 2
        q = 1 - p
        col0 = col_at(i)
        xbuf, bbuf, mbuf = xbufs[p], bbufs[p], mbufs[p]
        h_in[p][0].wait()

        for r, src in enumerate(BRIDGE_SRC):
            for k in range(VECS):
                bbuf[r, pl.ds(k * LANES, LANES)] = xbuf[src, pl.ds(k * LANES, LANES)]
        for r, src in enumerate(MISC_SRC):
            for k in range(VECS):
                mbuf[r, pl.ds(k * LANES, LANES)] = xbuf[src, pl.ds(k * LANES, LANES)]

        dst_cols = pl.ds(col0, CHUNK)
        h1 = pltpu.async_copy(
            xbuf.at[pl.ds(5, 152)], cards_hbm.at[pl.ds(0, 152), dst_cols], s_out[p])
        h2 = pltpu.async_copy(bbuf, cards_hbm.at[pl.ds(152, 8), dst_cols], s_out[p])
        h3 = pltpu.async_copy(mbuf, misc_hbm.at[:, dst_cols], s_out[p])
        h_in[p][1].wait()
        h4 = pltpu.async_copy(
            xbuf.at[pl.ds(175, 256)], cards_hbm.at[pl.ds(160, 256), dst_cols], s_out[p])
        h_out[p] = (h1, h2, h3, h4)
        if 2 <= i + 1 < N_CHUNKS:
            for h in h_out[q]:
                h.wait()
            h_in[q] = stage_in(i + 1, q)

    for p in range(2):
        for h in h_out[p]:
            h.wait()


@jax.jit
def kernel(inputs):
    x = inputs.T
    mesh = plsc.VectorSubcoreMesh(core_axis_name="c", subcore_axis_name="s")
    run = pl.kernel(
        _split_kernel,
        out_type=(
            jax.ShapeDtypeStruct((MISC_W, N_ROWS), jnp.float32),
            jax.ShapeDtypeStruct((CARD_W, N_ROWS), jnp.float32),
        ),
        mesh=mesh,
        scratch_types=[
            pltpu.VMEM((N_COLS, CHUNK), jnp.float32),
            pltpu.VMEM((N_COLS, CHUNK), jnp.float32),
            pltpu.VMEM((8, CHUNK), jnp.float32),
            pltpu.VMEM((8, CHUNK), jnp.float32),
            pltpu.VMEM((MISC_W, CHUNK), jnp.float32),
            pltpu.VMEM((MISC_W, CHUNK), jnp.float32),
            pltpu.SemaphoreType.DMA,
            pltpu.SemaphoreType.DMA,
            pltpu.SemaphoreType.DMA,
            pltpu.SemaphoreType.DMA,
            pltpu.SemaphoreType.DMA,
            pltpu.SemaphoreType.DMA,
        ],
        compiler_params=pltpu.CompilerParams(needs_layout_passes=False),
    )
    misc_t, cards_t = run(x)
    return misc_t.T, cards_t.T


# speedup vs baseline: 2.9483x; 1.0035x over previous
import jax
import jax.numpy as jnp
from jax import lax
from jax.experimental import pallas as pl
from jax.experimental.pallas import tpu as pltpu
from jax.experimental.pallas import tpu_sc as plsc

N_ROWS = 16384
N_COLS = 431
MISC_W = 15
CARD_W = 416

NUM_CORES = 2
NUM_SUBCORES = 16
NUM_WORKERS = NUM_CORES * NUM_SUBCORES
LANES = 16

COLS_PER_TILE = N_ROWS // NUM_WORKERS
CHUNK = 128
N_CHUNKS = COLS_PER_TILE // CHUNK
VECS = CHUNK // LANES

BRIDGE_SRC = (157, 158, 159, 160, 171, 172, 173, 174)
MISC_SRC = tuple(range(5)) + tuple(range(161, 171))


HALF = 216


def _split_kernel(x_hbm, misc_hbm, cards_hbm,
                  xb0, xb1, bb0, bb1, mb0, mb1,
                  s_a0, s_a1, s_b0, s_b1, s_o0, s_o1):
    tid = lax.axis_index("s") * NUM_CORES + lax.axis_index("c")

    def col_at(i):
        return pl.multiple_of(tid * COLS_PER_TILE + i * CHUNK, CHUNK)

    xbufs = (xb0, xb1)
    bbufs = (bb0, bb1)
    mbufs = (mb0, mb1)
    s_a = (s_a0, s_a1)
    s_b = (s_b0, s_b1)
    s_out = (s_o0, s_o1)

    def stage_in(i, p):
        cols = pl.ds(col_at(i), CHUNK)
        return (
            pltpu.async_copy(
                x_hbm.at[pl.ds(0, HALF), cols], xbufs[p].at[pl.ds(0, HALF)], s_a[p]),
            pltpu.async_copy(
                x_hbm.at[pl.ds(HALF, N_COLS - HALF), cols],
                xbufs[p].at[pl.ds(HALF, N_COLS - HALF)], s_b[p]),
        )

    h_in = [stage_in(0, 0), stage_in(1, 1)]
    h_out = [None, None]

    for i in range(N_CHUNKS):
        p = i % 2
        q = 1 - p
        col0 = col_at(i)
        xbuf, bbuf, mbuf = xbufs[p], bbufs[p], mbufs[p]
        h_in[p][0].wait()

        dst_cols = pl.ds(col0, CHUNK)
        h1 = pltpu.async_copy(
            xbuf.at[pl.ds(5, 152)], cards_hbm.at[pl.ds(0, 152), dst_cols], s_out[p])

        for r, src in enumerate(BRIDGE_SRC):
            for k in range(VECS):
                bbuf[r, pl.ds(k * LANES, LANES)] = xbuf[src, pl.ds(k * LANES, LANES)]
        for r, src in enumerate(MISC_SRC):
            for k in range(VECS):
                mbuf[r, pl.ds(k * LANES, LANES)] = xbuf[src, pl.ds(k * LANES, LANES)]

        h2 = pltpu.async_copy(bbuf, cards_hbm.at[pl.ds(152, 8), dst_cols], s_out[p])
        h3 = pltpu.async_copy(mbuf, misc_hbm.at[:, dst_cols], s_out[p])
        h_in[p][1].wait()
        h4 = pltpu.async_copy(
            xbuf.at[pl.ds(175, 256)], cards_hbm.at[pl.ds(160, 256), dst_cols], s_out[p])
        h_out[p] = (h1, h2, h3, h4)
        if 2 <= i + 1 < N_CHUNKS:
            for h in h_out[q]:
                h.wait()
            h_in[q] = stage_in(i + 1, q)

    for p in range(2):
        for h in h_out[p]:
            h.wait()


@jax.jit
def kernel(inputs):
    x = inputs.T
    mesh = plsc.VectorSubcoreMesh(core_axis_name="c", subcore_axis_name="s")
    run = pl.kernel(
        _split_kernel,
        out_type=(
            jax.ShapeDtypeStruct((MISC_W, N_ROWS), jnp.float32),
            jax.ShapeDtypeStruct((CARD_W, N_ROWS), jnp.float32),
        ),
        mesh=mesh,
        scratch_types=[
            pltpu.VMEM((N_COLS, CHUNK), jnp.float32),
            pltpu.VMEM((N_COLS, CHUNK), jnp.float32),
            pltpu.VMEM((8, CHUNK), jnp.float32),
            pltpu.VMEM((8, CHUNK), jnp.float32),
            pltpu.VMEM((MISC_W, CHUNK), jnp.float32),
            pltpu.VMEM((MISC_W, CHUNK), jnp.float32),
            pltpu.SemaphoreType.DMA,
            pltpu.SemaphoreType.DMA,
            pltpu.SemaphoreType.DMA,
            pltpu.SemaphoreType.DMA,
            pltpu.SemaphoreType.DMA,
            pltpu.SemaphoreType.DMA,
        ],
        compiler_params=pltpu.CompilerParams(needs_layout_passes=False),
    )
    misc_t, cards_t = run(x)
    return misc_t.T, cards_t.T
